# baseline (speedup 1.0000x reference)
import functools

import jax
import jax.numpy as jnp
from jax import lax
from jax.experimental import pallas as pl
from jax.experimental.pallas import tpu as pltpu

try:
    from jax.experimental.pallas import tpu_sc as plsc
except ImportError:
    plsc = None

DC_K = 8
CHUNK = 128
BLK_M = 16384
NEG = float("-inf")
EPS = 1e-6


def _cmax_body(q_ref, k_ref, out_ref, *, m_total, blk_m):
    g = pl.program_id(0)
    q = q_ref[...]
    qn = q / (jnp.sqrt(jnp.sum(q * q, axis=1, keepdims=True)) + EPS)
    k = k_ref[...]
    kn = k / (jnp.sqrt(jnp.sum(k * k, axis=1, keepdims=True)) + EPS)
    s = lax.dot_general(qn, kn, (((1,), (1,)), ((), ())),
                        preferred_element_type=jnp.float32)
    col = lax.broadcasted_iota(jnp.int32, (1, blk_m), 1)
    valid = m_total - g * blk_m
    s = jnp.where(col < valid, s, NEG)
    b = s.shape[0]
    cm = jnp.max(s.reshape(b, blk_m // CHUNK, CHUNK), axis=-1)
    out_ref[...] = cm


def _chunk_max(q, key_bank):
    b, d = q.shape
    m = key_bank.shape[0]
    nblk = -(-m // BLK_M)
    cpb = BLK_M // CHUNK
    ncp = nblk * cpb
    return pl.pallas_call(
        functools.partial(_cmax_body, m_total=m, blk_m=BLK_M),
        grid=(nblk,),
        in_specs=[
            pl.BlockSpec((b, d), lambda g: (0, 0)),
            pl.BlockSpec((BLK_M, d), lambda g: (g, 0)),
        ],
        out_specs=pl.BlockSpec((b, cpb), lambda g: (0, g)),
        out_shape=jax.ShapeDtypeStruct((b, ncp), jnp.float32),
    )(q, key_bank)


def _topchunk_body(cmax_ref, ids_ref, *, nchunks):
    v = cmax_ref[...]
    b, ncp = v.shape
    col = lax.broadcasted_iota(jnp.int32, (1, ncp), 1)
    v = jnp.where(col < nchunks, v, NEG)
    iota = lax.broadcasted_iota(jnp.int32, (b, ncp), 1)
    cols = []
    for _ in range(DC_K):
        mx = jnp.max(v, axis=1, keepdims=True)
        pos = jnp.min(jnp.where(v == mx, iota, ncp), axis=1, keepdims=True)
        cols.append(pos)
        v = jnp.where(iota == pos, NEG, v)
    ids = jnp.concatenate(cols, axis=1)
    ids_ref[...] = ids[:, None, :]


def _top_chunks(cmax, nchunks):
    b, ncp = cmax.shape
    out = pl.pallas_call(
        functools.partial(_topchunk_body, nchunks=nchunks),
        in_specs=[pl.BlockSpec((b, ncp), lambda: (0, 0))],
        out_specs=pl.BlockSpec((b, 1, DC_K), lambda: (0, 0, 0)),
        out_shape=jax.ShapeDtypeStruct((b, 1, DC_K), jnp.int32),
    )(cmax)
    return out.reshape(b, DC_K)


def _rescore_body(ids_ref, q_ref, *rest, m_total):
    krefs = rest[:DC_K]
    topi_ref = rest[DC_K]
    bq = pl.program_id(0)
    qb = q_ref[pl.ds(bq, 1), :]
    qn = qb / (jnp.sqrt(jnp.sum(qb * qb, axis=1, keepdims=True)) + EPS)
    pieces, gids = [], []
    lane = lax.broadcasted_iota(jnp.int32, (1, CHUNK), 1)
    for j in range(DC_K):
        kb = krefs[j][...]
        kn = kb / (jnp.sqrt(jnp.sum(kb * kb, axis=1, keepdims=True)) + EPS)
        s = lax.dot_general(qn, kn, (((1,), (1,)), ((), ())),
                            preferred_element_type=jnp.float32)
        gid = ids_ref[bq, j] * CHUNK + lane
        pieces.append(jnp.where(gid < m_total, s, NEG))
        gids.append(gid)
    s = jnp.concatenate(pieces, axis=1)
    g = jnp.concatenate(gids, axis=1)
    n = s.shape[1]
    iota = lax.broadcasted_iota(jnp.int32, (1, n), 1)
    cols = []
    for _ in range(DC_K):
        mx = jnp.max(s, axis=1, keepdims=True)
        pos = jnp.min(jnp.where(s == mx, iota, n), axis=1, keepdims=True)
        sel = iota == pos
        cols.append(jnp.sum(jnp.where(sel, g, 0), axis=1, keepdims=True))
        s = jnp.where(sel, NEG, s)
    topi = jnp.concatenate(cols, axis=1)
    topi_ref[...] = topi[:, None, :]


def _rescore(chunk_ids, q, key_bank):
    b, d = q.shape
    m = key_bank.shape[0]
    kspec = [
        pl.BlockSpec((CHUNK, d),
                     functools.partial(lambda j, bq, ids: (ids[bq, j], 0), j))
        for j in range(DC_K)
    ]
    grid_spec = pltpu.PrefetchScalarGridSpec(
        num_scalar_prefetch=1,
        grid=(b,),
        in_specs=[pl.BlockSpec((b, d), lambda bq, ids: (0, 0))] + kspec,
        out_specs=pl.BlockSpec((1, 1, DC_K), lambda bq, ids: (bq, 0, 0)),
    )
    out = pl.pallas_call(
        functools.partial(_rescore_body, m_total=m),
        grid_spec=grid_spec,
        out_shape=jax.ShapeDtypeStruct((b, 1, DC_K), jnp.int32),
    )(chunk_ids, q, *([key_bank] * DC_K))
    return out.reshape(b * DC_K)


def _gather_vals(val_bank, idx):
    n = idx.shape[0]
    d = val_bank.shape[1]
    info = plsc.get_sparse_core_info()
    nw = info.num_cores * info.num_subcores
    per_w = n // nw
    mesh = plsc.VectorSubcoreMesh(core_axis_name="c", subcore_axis_name="s")

    @functools.partial(
        pl.kernel,
        mesh=mesh,
        out_type=jax.ShapeDtypeStruct((n, d), jnp.float32),
        compiler_params=pltpu.CompilerParams(use_tc_tiling_on_sc=False),
        scratch_types=[
            pltpu.VMEM((per_w,), jnp.int32),
            pltpu.VMEM((per_w, d), jnp.float32),
            pltpu.SemaphoreType.DMA,
        ],
    )
    def gather(val_hbm, idx_hbm, out_hbm, idx_v, rows_v, sem):
        wid = lax.axis_index("s") * info.num_cores + lax.axis_index("c")
        base = wid * per_w
        pltpu.sync_copy(idx_hbm.at[pl.ds(base, per_w)], idx_v)
        pltpu.async_copy(val_hbm.at[idx_v], rows_v, sem).wait()
        pltpu.sync_copy(rows_v, out_hbm.at[pl.ds(base, per_w)])

    return gather(val_bank, idx)


def _project_body(v_ref, w_ref, out_ref):
    out_ref[...] = lax.dot_general(
        v_ref[...], w_ref[...], (((1,), (1,)), ((), ())),
        preferred_element_type=jnp.float32)


def _project(vals, w):
    n, d = vals.shape
    h = w.shape[0]
    return pl.pallas_call(
        _project_body,
        in_specs=[pl.BlockSpec((n, d), lambda: (0, 0)),
                  pl.BlockSpec((h, d), lambda: (0, 0))],
        out_specs=pl.BlockSpec((n, h), lambda: (0, 0)),
        out_shape=jax.ShapeDtypeStruct((n, h), jnp.float32),
    )(vals, w)


def kernel(q, key_bank, val_bank, W):
    b = q.shape[0]
    m = key_bank.shape[0]
    h = W.shape[0]
    nchunks = -(-m // CHUNK)
    cmax = _chunk_max(q, key_bank)
    chunk_ids = _top_chunks(cmax, nchunks)
    idx = _rescore(chunk_ids, q, key_bank)
    vals = _gather_vals(val_bank, idx)
    out = _project(vals, W)
    return out.reshape(b, DC_K, h)


# speedup vs baseline: 1.2691x; 1.2691x over previous
import functools

import jax
import jax.numpy as jnp
from jax import lax
from jax.experimental import pallas as pl
from jax.experimental.pallas import tpu as pltpu

try:
    from jax.experimental.pallas import tpu_sc as plsc
except ImportError:
    plsc = None

DC_K = 8
CHUNK = 128
BLK_M = 16384
NEG = float("-inf")
EPS = 1e-6


def _cmax_body(q_ref, k_ref, out_ref, *, m_total, blk_m):
    g = pl.program_id(0)
    q = q_ref[...]
    qn = q / (jnp.sqrt(jnp.sum(q * q, axis=1, keepdims=True)) + EPS)
    k = k_ref[...]
    d = k.shape[1]
    raw = lax.dot_general(qn, k, (((1,), (1,)), ((), ())),
                          preferred_element_type=jnp.float32)
    ksq = k * k
    ones = jnp.ones((1, d), dtype=jnp.float32)
    n2 = lax.dot_general(ones, ksq, (((1,), (1,)), ((), ())),
                         preferred_element_type=jnp.float32)
    inv = 1.0 / (jnp.sqrt(n2) + EPS)
    s = raw * inv
    col = lax.broadcasted_iota(jnp.int32, (1, blk_m), 1)
    valid = m_total - g * blk_m
    s = jnp.where(col < valid, s, NEG)
    b = s.shape[0]
    cm = jnp.max(s.reshape(b, blk_m // CHUNK, CHUNK), axis=-1)
    out_ref[...] = cm


def _chunk_max(q, key_bank):
    b, d = q.shape
    m = key_bank.shape[0]
    nblk = -(-m // BLK_M)
    cpb = BLK_M // CHUNK
    ncp = nblk * cpb
    return pl.pallas_call(
        functools.partial(_cmax_body, m_total=m, blk_m=BLK_M),
        grid=(nblk,),
        in_specs=[
            pl.BlockSpec((b, d), lambda g: (0, 0)),
            pl.BlockSpec((BLK_M, d), lambda g: (g, 0)),
        ],
        out_specs=pl.BlockSpec((b, cpb), lambda g: (0, g)),
        out_shape=jax.ShapeDtypeStruct((b, ncp), jnp.float32),
    )(q, key_bank)


def _topchunk_body(cmax_ref, ids_ref, *, nchunks):
    v = cmax_ref[...]
    b, ncp = v.shape
    col = lax.broadcasted_iota(jnp.int32, (1, ncp), 1)
    v = jnp.where(col < nchunks, v, NEG)
    iota = lax.broadcasted_iota(jnp.int32, (b, ncp), 1)
    cols = []
    for _ in range(DC_K):
        mx = jnp.max(v, axis=1, keepdims=True)
        pos = jnp.min(jnp.where(v == mx, iota, ncp), axis=1, keepdims=True)
        cols.append(pos)
        v = jnp.where(iota == pos, NEG, v)
    ids = jnp.concatenate(cols, axis=1)
    ids_ref[...] = ids[:, None, :]


def _top_chunks(cmax, nchunks):
    b, ncp = cmax.shape
    out = pl.pallas_call(
        functools.partial(_topchunk_body, nchunks=nchunks),
        in_specs=[pl.BlockSpec((b, ncp), lambda: (0, 0))],
        out_specs=pl.BlockSpec((b, 1, DC_K), lambda: (0, 0, 0)),
        out_shape=jax.ShapeDtypeStruct((b, 1, DC_K), jnp.int32),
    )(cmax)
    return out.reshape(b, DC_K)


def _rescore_body(ids_ref, q_ref, *rest, m_total):
    krefs = rest[:DC_K]
    topi_ref = rest[DC_K]
    bq = pl.program_id(0)
    qb = q_ref[pl.ds(bq, 1), :]
    qn = qb / (jnp.sqrt(jnp.sum(qb * qb, axis=1, keepdims=True)) + EPS)
    pieces, gids = [], []
    lane = lax.broadcasted_iota(jnp.int32, (1, CHUNK), 1)
    for j in range(DC_K):
        kb = krefs[j][...]
        kn = kb / (jnp.sqrt(jnp.sum(kb * kb, axis=1, keepdims=True)) + EPS)
        s = lax.dot_general(qn, kn, (((1,), (1,)), ((), ())),
                            preferred_element_type=jnp.float32)
        gid = ids_ref[bq, j] * CHUNK + lane
        pieces.append(jnp.where(gid < m_total, s, NEG))
        gids.append(gid)
    s = jnp.concatenate(pieces, axis=1)
    g = jnp.concatenate(gids, axis=1)
    n = s.shape[1]
    iota = lax.broadcasted_iota(jnp.int32, (1, n), 1)
    cols = []
    for _ in range(DC_K):
        mx = jnp.max(s, axis=1, keepdims=True)
        pos = jnp.min(jnp.where(s == mx, iota, n), axis=1, keepdims=True)
        sel = iota == pos
        cols.append(jnp.sum(jnp.where(sel, g, 0), axis=1, keepdims=True))
        s = jnp.where(sel, NEG, s)
    topi = jnp.concatenate(cols, axis=1)
    topi_ref[...] = topi[:, None, :]


def _rescore(chunk_ids, q, key_bank):
    b, d = q.shape
    m = key_bank.shape[0]
    kspec = [
        pl.BlockSpec((CHUNK, d),
                     functools.partial(lambda j, bq, ids: (ids[bq, j], 0), j))
        for j in range(DC_K)
    ]
    grid_spec = pltpu.PrefetchScalarGridSpec(
        num_scalar_prefetch=1,
        grid=(b,),
        in_specs=[pl.BlockSpec((b, d), lambda bq, ids: (0, 0))] + kspec,
        out_specs=pl.BlockSpec((1, 1, DC_K), lambda bq, ids: (bq, 0, 0)),
    )
    out = pl.pallas_call(
        functools.partial(_rescore_body, m_total=m),
        grid_spec=grid_spec,
        out_shape=jax.ShapeDtypeStruct((b, 1, DC_K), jnp.int32),
    )(chunk_ids, q, *([key_bank] * DC_K))
    return out.reshape(b * DC_K)


def _gather_project_body(blk_ref, rem_ref, *rest):
    vrefs = rest[:DC_K]
    w_ref = rest[DC_K]
    out_ref = rest[DC_K + 1]
    bq = pl.program_id(0)
    rows = [vrefs[j][pl.ds(rem_ref[bq, j], 1), :] for j in range(DC_K)]
    vals = jnp.concatenate(rows, axis=0)
    proj = lax.dot_general(vals, w_ref[...], (((1,), (1,)), ((), ())),
                           preferred_element_type=jnp.float32)
    out_ref[...] = proj[None]


def _gather_project(idx, val_bank, w):
    m, d = val_bank.shape
    h = w.shape[0]
    b = idx.shape[0] // DC_K
    blk = (idx // 8).reshape(b, DC_K)
    rem = (idx % 8).reshape(b, DC_K)
    vspec = [
        pl.BlockSpec((8, d),
                     functools.partial(lambda j, bq, bp, rp: (bp[bq, j], 0), j))
        for j in range(DC_K)
    ]
    grid_spec = pltpu.PrefetchScalarGridSpec(
        num_scalar_prefetch=2,
        grid=(b,),
        in_specs=vspec + [pl.BlockSpec((h, d), lambda bq, bp, rp: (0, 0))],
        out_specs=pl.BlockSpec((1, DC_K, h), lambda bq, bp, rp: (bq, 0, 0)),
    )
    return pl.pallas_call(
        _gather_project_body,
        grid_spec=grid_spec,
        out_shape=jax.ShapeDtypeStruct((b, DC_K, h), jnp.float32),
    )(blk, rem, *([val_bank] * DC_K), w)


def kernel(q, key_bank, val_bank, W):
    m = key_bank.shape[0]
    nchunks = -(-m // CHUNK)
    cmax = _chunk_max(q, key_bank)
    chunk_ids = _top_chunks(cmax, nchunks)
    idx = _rescore(chunk_ids, q, key_bank)
    return _gather_project(idx, val_bank, W)
